# baseline (speedup 1.0000x reference)
import functools

import jax
import jax.numpy as jnp
from jax import lax
from jax.experimental import pallas as pl
from jax.experimental.pallas import tpu as pltpu
from jax.experimental.pallas import tpu_sc as plsc

NC = 2
NS = 16
NW = NC * NS
C = 128
BLK = 8


def _sc_body(ctx, x_hbm, src_hbm, dst_hbm, sums_hbm, cnts_hbm,
             src_v0, src_v1, dst_v0, dst_v1, rows0, rows1, ones_v,
             acc_sh, cnt_sh, sem_i0, sem_i1, sem_g0, sem_g1):
    n_pad, rpt, ca, cb = ctx
    cid = lax.axis_index("c")
    sid = lax.axis_index("s")
    my_chunks = jnp.where(cid == 0, ca, cb)

    zeros16 = jnp.zeros((16,), jnp.float32)
    ones16 = jnp.ones((16,), jnp.float32)

    @pl.loop(0, C * 8)
    def _(t):
        rows0[t // 8, pl.ds(16 * (t % 8), 16)] = zeros16

    @pl.loop(0, C)
    def _(i):
        ones_v[i, :] = zeros16

    row0 = sid * rpt

    @pl.loop(0, rpt // C)
    def _(j):
        pltpu.sync_copy(rows0, acc_sh.at[pl.ds(row0 + j * C, C)])
        pltpu.sync_copy(ones_v, cnt_sh.at[pl.ds(row0 + j * C, C)])

    @pl.loop(0, C)
    def _(i):
        ones_v[i, :] = ones16

    plsc.subcore_barrier()

    base_w = jnp.where(cid == 0, sid * (ca * C),
                       NS * ca * C + sid * (cb * C))
    srcs = (src_v0, src_v1)
    dsts = (dst_v0, dst_v1)
    rows = (rows0, rows1)
    sem_i = (sem_i0, sem_i1)
    sem_g = (sem_g0, sem_g1)

    def idx_issue(k, b):
        pltpu.async_copy(src_hbm.at[pl.ds(base_w + k * C, C)], srcs[b],
                         sem_i[b])
        pltpu.async_copy(dst_hbm.at[pl.ds(base_w + k * C, C)], dsts[b],
                         sem_i[b])

    def idx_wait(k, b):
        pltpu.make_async_copy(src_hbm.at[pl.ds(base_w + k * C, C)], srcs[b],
                              sem_i[b]).wait()
        pltpu.make_async_copy(dst_hbm.at[pl.ds(base_w + k * C, C)], dsts[b],
                              sem_i[b]).wait()

    def g_issue(b):
        pltpu.async_copy(x_hbm.at[srcs[b]], rows[b], sem_g[b])

    def g_wait(b):
        pltpu.make_async_copy(x_hbm.at[srcs[b]], rows[b], sem_g[b]).wait()

    def scat(b):
        pltpu.sync_copy(rows[b], acc_sh.at[dsts[b]], add=True)
        pltpu.sync_copy(ones_v, cnt_sh.at[dsts[b]], add=True)

    pltpu.sync_copy(src_hbm.at[pl.ds(base_w, C)], srcs[0])
    pltpu.sync_copy(dst_hbm.at[pl.ds(base_w, C)], dsts[0])
    g_issue(0)
    idx_issue(1, 1)

    n_pairs = (my_chunks - 1) // 2

    @pl.loop(0, n_pairs)
    def _(m):
        k0 = 2 * m
        g_wait(0)
        idx_wait(k0 + 1, 1)
        g_issue(1)
        scat(0)
        idx_issue(k0 + 2, 0)
        g_wait(1)
        idx_wait(k0 + 2, 0)
        g_issue(0)
        scat(1)

        @pl.when(m < n_pairs - 1)
        def _():
            idx_issue(k0 + 3, 1)

    g_wait(0)
    scat(0)

    plsc.subcore_barrier()

    pltpu.sync_copy(acc_sh.at[pl.ds(row0, rpt)],
                    sums_hbm.at[cid, pl.ds(row0, rpt)])
    pltpu.sync_copy(cnt_sh.at[pl.ds(row0, rpt)],
                    cnts_hbm.at[cid, pl.ds(row0, rpt)])


CA_FRAC = 0.95


def _split_chunks(total):
    ca = int(total * CA_FRAC / NS)
    ca = ca if ca % 2 == 1 else ca - 1
    cb = total // NS - ca
    assert cb % 2 == 1 and ca >= 1 and cb >= 1
    return ca, cb


def _segment_sums(x, src3, dst3, n_pad):
    ca, cb = _split_chunks(src3.shape[0] // C)
    rpt = n_pad // NS
    d = x.shape[1]
    mesh = plsc.VectorSubcoreMesh(core_axis_name="c", subcore_axis_name="s")
    body = functools.partial(_sc_body, (n_pad, rpt, ca, cb))
    return pl.kernel(
        body,
        out_type=(
            jax.ShapeDtypeStruct((NC, n_pad, d), jnp.float32),
            jax.ShapeDtypeStruct((NC, n_pad, 16), jnp.float32),
        ),
        mesh=mesh,
        compiler_params=pltpu.CompilerParams(use_tc_tiling_on_sc=False),
        scratch_types=[
            pltpu.VMEM((C,), jnp.int32),
            pltpu.VMEM((C,), jnp.int32),
            pltpu.VMEM((C,), jnp.int32),
            pltpu.VMEM((C,), jnp.int32),
            pltpu.VMEM((C, d), jnp.float32),
            pltpu.VMEM((C, d), jnp.float32),
            pltpu.VMEM((C, 16), jnp.float32),
            pltpu.VMEM_SHARED((n_pad, d), jnp.float32),
            pltpu.VMEM_SHARED((n_pad, 16), jnp.float32),
            pltpu.SemaphoreType.DMA,
            pltpu.SemaphoreType.DMA,
            pltpu.SemaphoreType.DMA,
            pltpu.SemaphoreType.DMA,
        ],
    )(x, src3, dst3)


def _tc_body(s_ref, c_ref, w_ref, b_ref, o_ref):
    s = s_ref[0] + s_ref[1]
    c = c_ref[0, :, 0:1] + c_ref[1, :, 0:1]
    mean = s / jnp.maximum(c, 1.0)
    h = lax.dot_general(mean, w_ref[...], (((1,), (1,)), ((), ())),
                        preferred_element_type=jnp.float32)
    out = h + b_ref[...] * jnp.minimum(c, 1.0)
    o_ref[...] = jnp.maximum(out, 0.0)


def _finish(sums, cnts, W, b, n_pad, rows_blk):
    d_in = W.shape[1]
    d_out = W.shape[0]
    grid = (n_pad // rows_blk,)
    return pl.pallas_call(
        _tc_body,
        grid=grid,
        in_specs=[
            pl.BlockSpec((NC, rows_blk, d_in), lambda i: (0, i, 0)),
            pl.BlockSpec((NC, rows_blk, 16), lambda i: (0, i, 0)),
            pl.BlockSpec((d_out, d_in), lambda i: (0, 0)),
            pl.BlockSpec((1, d_out), lambda i: (0, 0)),
        ],
        out_specs=pl.BlockSpec((rows_blk, d_out), lambda i: (i, 0)),
        out_shape=jax.ShapeDtypeStruct((n_pad, d_out), jnp.float32),
    )(sums, cnts, W, b.reshape(1, d_out))


def kernel(x, edge_index, W, b):
    n = x.shape[0]
    e = edge_index.shape[1]

    n_pad = ((n + 1) + NS * C - 1) // (NS * C) * (NS * C)
    egrp = NW * C
    e_pad = (e + egrp - 1) // egrp * egrp

    src = edge_index[0].astype(jnp.int32)
    dst = edge_index[1].astype(jnp.int32)
    pad = e_pad - e
    if pad:
        src = jnp.concatenate([src, jnp.zeros((pad,), jnp.int32)])
        dst = jnp.concatenate(
            [dst, n + (jnp.arange(pad, dtype=jnp.int32) % C)])
    sums, cnts = _segment_sums(x, src, dst, n_pad)
    out = _finish(sums, cnts, W, b, n_pad, rows_blk=1024)
    return out[:n]


# speedup vs baseline: 1.0600x; 1.0600x over previous
import functools

import jax
import jax.numpy as jnp
from jax import lax
from jax.experimental import pallas as pl
from jax.experimental.pallas import tpu as pltpu
from jax.experimental.pallas import tpu_sc as plsc

NC = 2
NS = 16
NW = NC * NS
C = 128
BLK = 8


def _sc_body(ctx, x_hbm, src_hbm, dst_hbm, sums_hbm, cnts_hbm,
             src_v0, src_v1, dst_v0, dst_v1, rows0, rows1, ones_v,
             acc_sh, cnt_sh, sem_i0, sem_i1, sem_g0, sem_g1):
    n_pad, rpt, ca, cb = ctx
    cid = lax.axis_index("c")
    sid = lax.axis_index("s")
    my_chunks = jnp.where(cid == 0, ca, cb)

    zeros16 = jnp.zeros((16,), jnp.float32)
    ones16 = jnp.ones((16,), jnp.float32)

    @pl.loop(0, C * 8)
    def _(t):
        rows0[t // 8, pl.ds(16 * (t % 8), 16)] = zeros16

    @pl.loop(0, C)
    def _(i):
        ones_v[i, :] = zeros16

    row0 = sid * rpt

    @pl.loop(0, rpt // C)
    def _(j):
        pltpu.sync_copy(rows0, acc_sh.at[pl.ds(row0 + j * C, C)])
        pltpu.sync_copy(ones_v, cnt_sh.at[pl.ds(row0 + j * C, C)])

    @pl.loop(0, C)
    def _(i):
        ones_v[i, :] = ones16

    plsc.subcore_barrier()

    base_w = jnp.where(cid == 0, sid * (ca * C),
                       NS * ca * C + sid * (cb * C))
    srcs = (src_v0, src_v1)
    dsts = (dst_v0, dst_v1)
    rows = (rows0, rows1)
    sem_i = (sem_i0, sem_i1)
    sem_g = (sem_g0, sem_g1)

    def idx_issue(k, b):
        pltpu.async_copy(src_hbm.at[pl.ds(base_w + k * C, C)], srcs[b],
                         sem_i[b])
        pltpu.async_copy(dst_hbm.at[pl.ds(base_w + k * C, C)], dsts[b],
                         sem_i[b])

    def idx_wait(k, b):
        pltpu.make_async_copy(src_hbm.at[pl.ds(base_w + k * C, C)], srcs[b],
                              sem_i[b]).wait()
        pltpu.make_async_copy(dst_hbm.at[pl.ds(base_w + k * C, C)], dsts[b],
                              sem_i[b]).wait()

    def g_issue(b):
        pltpu.async_copy(x_hbm.at[srcs[b]], rows[b], sem_g[b])

    def g_wait(b):
        pltpu.make_async_copy(x_hbm.at[srcs[b]], rows[b], sem_g[b]).wait()

    def scat(b):
        pltpu.sync_copy(rows[b], acc_sh.at[dsts[b]], add=True)
        pltpu.sync_copy(ones_v, cnt_sh.at[dsts[b]], add=True)

    pltpu.sync_copy(src_hbm.at[pl.ds(base_w, C)], srcs[0])
    pltpu.sync_copy(dst_hbm.at[pl.ds(base_w, C)], dsts[0])
    g_issue(0)
    idx_issue(1, 1)

    n_pairs = (my_chunks - 1) // 2

    @pl.loop(0, n_pairs)
    def _(m):
        k0 = 2 * m
        g_wait(0)
        idx_wait(k0 + 1, 1)
        g_issue(1)
        scat(0)
        idx_issue(k0 + 2, 0)
        g_wait(1)
        idx_wait(k0 + 2, 0)
        g_issue(0)
        scat(1)

        @pl.when(m < n_pairs - 1)
        def _():
            idx_issue(k0 + 3, 1)

    g_wait(0)
    scat(0)

    plsc.subcore_barrier()

    pltpu.sync_copy(acc_sh.at[pl.ds(row0, rpt)],
                    sums_hbm.at[cid, pl.ds(row0, rpt)])
    pltpu.sync_copy(cnt_sh.at[pl.ds(row0, rpt)],
                    cnts_hbm.at[cid, pl.ds(row0, rpt)])


CA_FRAC = 0.92


def _split_chunks(total):
    ca = int(total * CA_FRAC / NS)
    ca = ca if ca % 2 == 1 else ca - 1
    cb = total // NS - ca
    assert cb % 2 == 1 and ca >= 1 and cb >= 1
    return ca, cb


def _segment_sums(x, src3, dst3, n_pad):
    ca, cb = _split_chunks(src3.shape[0] // C)
    rpt = n_pad // NS
    d = x.shape[1]
    mesh = plsc.VectorSubcoreMesh(core_axis_name="c", subcore_axis_name="s")
    body = functools.partial(_sc_body, (n_pad, rpt, ca, cb))
    return pl.kernel(
        body,
        out_type=(
            jax.ShapeDtypeStruct((NC, n_pad, d), jnp.float32),
            jax.ShapeDtypeStruct((NC, n_pad, 16), jnp.float32),
        ),
        mesh=mesh,
        compiler_params=pltpu.CompilerParams(use_tc_tiling_on_sc=False),
        scratch_types=[
            pltpu.VMEM((C,), jnp.int32),
            pltpu.VMEM((C,), jnp.int32),
            pltpu.VMEM((C,), jnp.int32),
            pltpu.VMEM((C,), jnp.int32),
            pltpu.VMEM((C, d), jnp.float32),
            pltpu.VMEM((C, d), jnp.float32),
            pltpu.VMEM((C, 16), jnp.float32),
            pltpu.VMEM_SHARED((n_pad, d), jnp.float32),
            pltpu.VMEM_SHARED((n_pad, 16), jnp.float32),
            pltpu.SemaphoreType.DMA,
            pltpu.SemaphoreType.DMA,
            pltpu.SemaphoreType.DMA,
            pltpu.SemaphoreType.DMA,
        ],
    )(x, src3, dst3)


def _tc_body(s_ref, c_ref, w_ref, b_ref, o_ref):
    s = s_ref[0] + s_ref[1]
    c = c_ref[0, :, 0:1] + c_ref[1, :, 0:1]
    mean = s / jnp.maximum(c, 1.0)
    h = lax.dot_general(mean, w_ref[...], (((1,), (1,)), ((), ())),
                        preferred_element_type=jnp.float32)
    out = h + b_ref[...] * jnp.minimum(c, 1.0)
    o_ref[...] = jnp.maximum(out, 0.0)


def _finish(sums, cnts, W, b, n_pad, rows_blk):
    d_in = W.shape[1]
    d_out = W.shape[0]
    grid = (n_pad // rows_blk,)
    return pl.pallas_call(
        _tc_body,
        grid=grid,
        in_specs=[
            pl.BlockSpec((NC, rows_blk, d_in), lambda i: (0, i, 0)),
            pl.BlockSpec((NC, rows_blk, 16), lambda i: (0, i, 0)),
            pl.BlockSpec((d_out, d_in), lambda i: (0, 0)),
            pl.BlockSpec((1, d_out), lambda i: (0, 0)),
        ],
        out_specs=pl.BlockSpec((rows_blk, d_out), lambda i: (i, 0)),
        out_shape=jax.ShapeDtypeStruct((n_pad, d_out), jnp.float32),
    )(sums, cnts, W, b.reshape(1, d_out))


def kernel(x, edge_index, W, b):
    n = x.shape[0]
    e = edge_index.shape[1]

    n_pad = ((n + 1) + NS * C - 1) // (NS * C) * (NS * C)
    egrp = NW * C
    e_pad = (e + egrp - 1) // egrp * egrp

    src = edge_index[0].astype(jnp.int32)
    dst = edge_index[1].astype(jnp.int32)
    pad = e_pad - e
    if pad:
        src = jnp.concatenate([src, jnp.zeros((pad,), jnp.int32)])
        dst = jnp.concatenate(
            [dst, n + (jnp.arange(pad, dtype=jnp.int32) % C)])
    sums, cnts = _segment_sums(x, src, dst, n_pad)
    out = _finish(sums, cnts, W, b, n_pad, rows_blk=1024)
    return out[:n]


# speedup vs baseline: 1.0818x; 1.0206x over previous
import functools

import jax
import jax.numpy as jnp
from jax import lax
from jax.experimental import pallas as pl
from jax.experimental.pallas import tpu as pltpu
from jax.experimental.pallas import tpu_sc as plsc

NC = 2
NS = 16
NW = NC * NS
C = 128
BLK = 8


def _sc_body(ctx, x_hbm, src_hbm, dst_hbm, sums_hbm, cnts_hbm,
             src_v0, src_v1, dst_v0, dst_v1, rows0, rows1, ones_v,
             acc_sh, cnt_sh, sem_i0, sem_i1, sem_g0, sem_g1):
    n_pad, rpt, ca, cb = ctx
    cid = lax.axis_index("c")
    sid = lax.axis_index("s")
    my_chunks = jnp.where(cid == 0, ca, cb)

    zeros16 = jnp.zeros((16,), jnp.float32)
    ones16 = jnp.ones((16,), jnp.float32)

    @pl.loop(0, C * 8)
    def _(t):
        rows0[t // 8, pl.ds(16 * (t % 8), 16)] = zeros16

    @pl.loop(0, C)
    def _(i):
        ones_v[i, :] = zeros16

    row0 = sid * rpt

    @pl.loop(0, rpt // C)
    def _(j):
        pltpu.sync_copy(rows0, acc_sh.at[pl.ds(row0 + j * C, C)])
        pltpu.sync_copy(ones_v, cnt_sh.at[pl.ds(row0 + j * C, C)])

    @pl.loop(0, C)
    def _(i):
        ones_v[i, :] = ones16

    plsc.subcore_barrier()

    base_w = jnp.where(cid == 0, sid * (ca * C),
                       NS * ca * C + sid * (cb * C))
    srcs = (src_v0, src_v1)
    dsts = (dst_v0, dst_v1)
    rows = (rows0, rows1)
    sem_i = (sem_i0, sem_i1)
    sem_g = (sem_g0, sem_g1)

    def idx_issue(k, b):
        pltpu.async_copy(src_hbm.at[pl.ds(base_w + k * C, C)], srcs[b],
                         sem_i[b])
        pltpu.async_copy(dst_hbm.at[pl.ds(base_w + k * C, C)], dsts[b],
                         sem_i[b])

    def idx_wait(k, b):
        pltpu.make_async_copy(src_hbm.at[pl.ds(base_w + k * C, C)], srcs[b],
                              sem_i[b]).wait()
        pltpu.make_async_copy(dst_hbm.at[pl.ds(base_w + k * C, C)], dsts[b],
                              sem_i[b]).wait()

    def g_issue(b):
        pltpu.async_copy(x_hbm.at[srcs[b]], rows[b], sem_g[b])

    def g_wait(b):
        pltpu.make_async_copy(x_hbm.at[srcs[b]], rows[b], sem_g[b]).wait()

    def scat(b):
        pltpu.sync_copy(rows[b], acc_sh.at[dsts[b]], add=True)
        pltpu.sync_copy(ones_v, cnt_sh.at[dsts[b]], add=True)

    pltpu.sync_copy(src_hbm.at[pl.ds(base_w, C)], srcs[0])
    pltpu.sync_copy(dst_hbm.at[pl.ds(base_w, C)], dsts[0])
    g_issue(0)
    idx_issue(1, 1)

    n_pairs = (my_chunks - 1) // 2

    @pl.loop(0, n_pairs)
    def _(m):
        k0 = 2 * m
        g_wait(0)
        idx_wait(k0 + 1, 1)
        g_issue(1)
        scat(0)
        idx_issue(k0 + 2, 0)
        g_wait(1)
        idx_wait(k0 + 2, 0)
        g_issue(0)
        scat(1)

        @pl.when(m < n_pairs - 1)
        def _():
            idx_issue(k0 + 3, 1)

    g_wait(0)
    scat(0)

    plsc.subcore_barrier()

    pltpu.sync_copy(acc_sh.at[pl.ds(row0, rpt)],
                    sums_hbm.at[cid, pl.ds(row0, rpt)])
    pltpu.sync_copy(cnt_sh.at[pl.ds(row0, rpt)],
                    cnts_hbm.at[cid, pl.ds(row0, rpt)])


CA_FRAC = 0.9


def _split_chunks(total):
    ca = int(total * CA_FRAC / NS)
    ca = ca if ca % 2 == 1 else ca - 1
    cb = total // NS - ca
    assert cb % 2 == 1 and ca >= 1 and cb >= 1
    return ca, cb


def _segment_sums(x, src3, dst3, n_pad):
    ca, cb = _split_chunks(src3.shape[0] // C)
    rpt = n_pad // NS
    d = x.shape[1]
    mesh = plsc.VectorSubcoreMesh(core_axis_name="c", subcore_axis_name="s")
    body = functools.partial(_sc_body, (n_pad, rpt, ca, cb))
    return pl.kernel(
        body,
        out_type=(
            jax.ShapeDtypeStruct((NC, n_pad, d), jnp.float32),
            jax.ShapeDtypeStruct((NC, n_pad, 16), jnp.float32),
        ),
        mesh=mesh,
        compiler_params=pltpu.CompilerParams(use_tc_tiling_on_sc=False),
        scratch_types=[
            pltpu.VMEM((C,), jnp.int32),
            pltpu.VMEM((C,), jnp.int32),
            pltpu.VMEM((C,), jnp.int32),
            pltpu.VMEM((C,), jnp.int32),
            pltpu.VMEM((C, d), jnp.float32),
            pltpu.VMEM((C, d), jnp.float32),
            pltpu.VMEM((C, 16), jnp.float32),
            pltpu.VMEM_SHARED((n_pad, d), jnp.float32),
            pltpu.VMEM_SHARED((n_pad, 16), jnp.float32),
            pltpu.SemaphoreType.DMA,
            pltpu.SemaphoreType.DMA,
            pltpu.SemaphoreType.DMA,
            pltpu.SemaphoreType.DMA,
        ],
    )(x, src3, dst3)


def _tc_body(s_ref, c_ref, w_ref, b_ref, o_ref):
    s = s_ref[0] + s_ref[1]
    c = c_ref[0, :, 0:1] + c_ref[1, :, 0:1]
    mean = s / jnp.maximum(c, 1.0)
    h = lax.dot_general(mean, w_ref[...], (((1,), (1,)), ((), ())),
                        preferred_element_type=jnp.float32)
    out = h + b_ref[...] * jnp.minimum(c, 1.0)
    o_ref[...] = jnp.maximum(out, 0.0)


def _finish(sums, cnts, W, b, n_pad, rows_blk):
    d_in = W.shape[1]
    d_out = W.shape[0]
    grid = (n_pad // rows_blk,)
    return pl.pallas_call(
        _tc_body,
        grid=grid,
        in_specs=[
            pl.BlockSpec((NC, rows_blk, d_in), lambda i: (0, i, 0)),
            pl.BlockSpec((NC, rows_blk, 16), lambda i: (0, i, 0)),
            pl.BlockSpec((d_out, d_in), lambda i: (0, 0)),
            pl.BlockSpec((1, d_out), lambda i: (0, 0)),
        ],
        out_specs=pl.BlockSpec((rows_blk, d_out), lambda i: (i, 0)),
        out_shape=jax.ShapeDtypeStruct((n_pad, d_out), jnp.float32),
    )(sums, cnts, W, b.reshape(1, d_out))


def kernel(x, edge_index, W, b):
    n = x.shape[0]
    e = edge_index.shape[1]

    n_pad = ((n + 1) + NS * C - 1) // (NS * C) * (NS * C)
    egrp = NW * C
    e_pad = (e + egrp - 1) // egrp * egrp

    src = edge_index[0].astype(jnp.int32)
    dst = edge_index[1].astype(jnp.int32)
    pad = e_pad - e
    if pad:
        src = jnp.concatenate([src, jnp.zeros((pad,), jnp.int32)])
        dst = jnp.concatenate(
            [dst, n + (jnp.arange(pad, dtype=jnp.int32) % C)])
    sums, cnts = _segment_sums(x, src, dst, n_pad)
    out = _finish(sums, cnts, W, b, n_pad, rows_blk=1024)
    return out[:n]
